# baseline (speedup 1.0000x reference)
import jax
import jax.numpy as jnp
from jax import lax
from jax.experimental import pallas as pl
from jax.experimental.pallas import tpu as pltpu
from jax.experimental.pallas import tpu_sc as plsc

_L = 16
_C = 128
_NSC = 2
_NSUB = 16
_NW = _NSC * _NSUB
_G = 64

_SC_PARAMS = pltpu.CompilerParams(needs_layout_passes=False)
_SC_PARAMS_NT = pltpu.CompilerParams(
    needs_layout_passes=False, use_tc_tiling_on_sc=False)


def _mesh():
    return plsc.VectorSubcoreMesh(core_axis_name="c", subcore_axis_name="s")


def _sc_scalar_pass(dst2d, w2d, cpt, npad, nnodes, src2d=None, zs=None):
    per_tile = npad // _NSUB
    with_gather = zs is not None

    scratch = [
        pltpu.VMEM((cpt, _C), jnp.int32),
        pltpu.VMEM((cpt, _C), jnp.float32),
        pltpu.VMEM((per_tile,), jnp.float32),
        pltpu.VMEM_SHARED((npad,), jnp.float32),
        pltpu.SemaphoreType.DMA,
    ]
    if with_gather:
        scratch.insert(0, pltpu.VMEM((cpt, _C), jnp.int32))
        scratch.insert(1, pltpu.VMEM((nnodes,), jnp.float32))
        scratch.insert(2, pltpu.VMEM((cpt, _C), jnp.float32))

    def body(*refs):
        if with_gather:
            (src_hbm, dst_hbm, w_hbm, zs_hbm, out_hbm,
             srci, zsv, prod, dsti, wv, bnc, acc, sem) = refs
        else:
            (dst_hbm, w_hbm, out_hbm,
             dsti, wv, bnc, acc, sem) = refs
        cid = lax.axis_index("c")
        sid = lax.axis_index("s")
        wid = sid * _NSC + cid
        ofs = wid * cpt

        pltpu.sync_copy(dst_hbm.at[pl.ds(ofs, cpt), :], dsti)
        pltpu.sync_copy(w_hbm.at[pl.ds(ofs, cpt), :], wv)
        if with_gather:
            pltpu.sync_copy(src_hbm.at[pl.ds(ofs, cpt), :], srci)
            pltpu.sync_copy(zs_hbm, zsv)

        zero = jnp.zeros((_L,), jnp.float32)
        for i in range(per_tile // _L):
            bnc[pl.ds(i * _L, _L)] = zero
        pltpu.sync_copy(bnc, acc.at[pl.ds(sid * per_tile, per_tile)])

        if with_gather:
            def pc(c, _):
                for k in range(_C // _L):
                    sl = pl.ds(k * _L, _L)
                    idx = srci[c, sl]
                    prod[c, sl] = plsc.load_gather(zsv, [idx]) * wv[c, sl]
                return 0
            lax.fori_loop(0, cpt, pc, 0)

        plsc.subcore_barrier()

        val = prod if with_gather else wv

        descs = [
            pltpu.async_copy(val.at[c], acc.at[dsti.at[c]], sem, add=True)
            for c in range(cpt)
        ]
        for d in descs:
            d.wait()

        plsc.subcore_barrier()
        pltpu.sync_copy(acc.at[pl.ds(sid * per_tile, per_tile)], bnc)
        pltpu.sync_copy(
            bnc, out_hbm.at[pl.ds(cid * npad + sid * per_tile, per_tile)])

    kern = pl.kernel(
        body,
        mesh=_mesh(),
        out_type=jax.ShapeDtypeStruct((_NSC * npad,), jnp.float32),
        scratch_types=scratch,
        compiler_params=_SC_PARAMS,
    )
    if with_gather:
        return kern(src2d, dst2d, w2d, zs)
    return kern(dst2d, w2d)


_CH = 64


def _sc_row_pass(src2d, dst2d, w2d, xsb, cpt, nrows, feat):
    rows_per_tile = nrows // _NSUB
    blk = 16
    assert rows_per_tile % _CH == 0
    assert cpt % blk == 0

    def body(src_hbm, dst_hbm, w_hbm, xs_hbm, out_hbm,
             srci, dsti, wv, rb0, rb1, rf0, rf1, acc, g0, g1, s0, s1):
        rows_b = [rb0, rb1]
        rows_f = [rf0, rf1]
        gsem = [g0, g1]
        ssem = [s0, s1]
        cid = lax.axis_index("c")
        sid = lax.axis_index("s")
        wid = sid * _NSC + cid
        ofs = wid * cpt
        base = sid * rows_per_tile

        zero = jnp.zeros((_L,), jnp.float32)

        def zr(j, _):
            for k in range(feat // _L):
                rf0[j, pl.ds(k * _L, _L)] = zero
            return 0

        lax.fori_loop(0, _CH, zr, 0)

        def zb(i, _):
            pltpu.sync_copy(rf0, acc.at[pl.ds(base + i * _CH, _CH), :])
            return 0

        lax.fori_loop(0, rows_per_tile // _CH, zb, 0)
        plsc.subcore_barrier()

        def block(b, _):
            bofs = ofs + b * blk
            pltpu.sync_copy(src_hbm.at[pl.ds(bofs, blk), :], srci)
            pltpu.sync_copy(dst_hbm.at[pl.ds(bofs, blk), :], dsti)
            pltpu.sync_copy(w_hbm.at[pl.ds(bofs, blk), :], wv)
            gd = [None, None]
            sd = [None, None]
            gd[0] = pltpu.async_copy(
                xs_hbm.at[srci.at[0]], rows_b[0], gsem[0])
            for j in range(blk):
                p = j % 2
                q = 1 - p
                if j + 1 < blk:
                    gd[q] = pltpu.async_copy(
                        xs_hbm.at[srci.at[j + 1]], rows_b[q], gsem[q])
                gd[p].wait()
                if sd[p] is not None:
                    sd[p].wait()
                vj = jnp.full((_L,), j, jnp.int32)

                def scale(g, _):
                    e0 = g * _L
                    for jj in range(_L):
                        e = e0 + jj
                        wj = plsc.load_gather(
                            wv, [vj, jnp.full((_L,), e, jnp.int32)])
                        for k in range(feat // (2 * _L)):
                            v16 = rows_b[p][e, pl.ds(k * _L, _L)]
                            v32 = plsc.bitcast(v16, jnp.bfloat16)
                            a, bb = plsc.unpack(
                                v32, format=plsc.PackFormat.INTERLEAVED)
                            rows_f[p][e, pl.ds(k * 2 * _L, _L)] = a * wj
                            rows_f[p][e, pl.ds(k * 2 * _L + _L, _L)] = bb * wj
                    return 0

                lax.fori_loop(0, _CH // _L, scale, 0)
                sd[p] = pltpu.async_copy(
                    rows_f[p], acc.at[dsti.at[j]], ssem[p], add=True)
            for d in sd:
                d.wait()
            return 0

        lax.fori_loop(0, cpt // blk, block, 0)
        plsc.subcore_barrier()

        for k in range(rows_per_tile // _CH):
            sl = pl.ds(base + k * _CH, _CH)
            pltpu.sync_copy(acc.at[sl, :], rf0)
            pltpu.sync_copy(rf0, out_hbm.at[cid, sl, :])

    kern = pl.kernel(
        body,
        mesh=_mesh(),
        out_type=jax.ShapeDtypeStruct((_NSC, nrows, feat), jnp.float32),
        scratch_types=(
            [pltpu.VMEM((blk, _CH), jnp.int32),
             pltpu.VMEM((blk, _CH), jnp.int32),
             pltpu.VMEM((blk, _CH), jnp.float32)]
            + [pltpu.VMEM((_CH, feat // 2), jnp.float32) for _ in range(2)]
            + [pltpu.VMEM((_CH, feat), jnp.float32) for _ in range(2)]
            + [pltpu.VMEM_SHARED((nrows, feat), jnp.float32)]
            + [pltpu.SemaphoreType.DMA for _ in range(4)]
        ),
        compiler_params=_SC_PARAMS_NT,
    )
    n64 = src2d.size // _CH
    return kern(src2d.reshape(n64, _CH), dst2d.reshape(n64, _CH),
                w2d.reshape(n64, _CH), xsb)


def _bf16_perm(feat):
    perm = []
    for g in range(feat // 32):
        for i in range(16):
            perm.append(g * 32 + i)
            perm.append(g * 32 + 16 + i)
    return jnp.asarray(perm, dtype=jnp.int32)


def _tc_k1(x, W1, dega, degb):
    def body(x_ref, w_ref, da_ref, db_ref, xs_ref, dinv_ref, xsb_ref):
        deg = da_ref[...] + db_ref[...] + 1.0
        dinv = jnp.where(deg > 0,
                         lax.rsqrt(jnp.maximum(deg, 1e-12)),
                         0.0)
        xw = jnp.dot(x_ref[...], w_ref[...],
                     preferred_element_type=jnp.float32)
        xs = xw * dinv
        xs_ref[...] = xs
        dinv_ref[...] = dinv
        xsb_ref[...] = xs.astype(jnp.bfloat16)

    n, f = x.shape
    h = W1.shape[1]
    return pl.pallas_call(
        body,
        out_shape=[
            jax.ShapeDtypeStruct((n, h), jnp.float32),
            jax.ShapeDtypeStruct((n, 1), jnp.float32),
            jax.ShapeDtypeStruct((n, h), jnp.bfloat16),
        ],
    )(x, W1, dega, degb)


def _tc_k2(s1p, xs, dinv, b1r, W2, Wout):
    n = xs.shape[0]

    def body(s_ref, xs_ref, dinv_ref, b1_ref, w2_ref, wo_ref, zs_ref):
        u = jnp.dot(w2_ref[...], wo_ref[...],
                    preferred_element_type=jnp.float32)
        dinv = dinv_ref[...]
        s1 = s_ref[0, :n, :] + s_ref[1, :n, :]
        x1 = dinv * (s1 + xs_ref[...]) + b1_ref[...]
        x1a = 1.0 / (1.0 + jnp.exp(-x1))
        z = jnp.dot(x1a, u, preferred_element_type=jnp.float32)
        zs_ref[...] = z * dinv

    return pl.pallas_call(
        body,
        out_shape=jax.ShapeDtypeStruct((n, 1), jnp.float32),
    )(s1p, xs, dinv, b1r, W2, Wout)


def _tc_k3(ta, tb, zs, dinv, batch_row, b2r, woutr, bout2d):
    def body(ta_ref, tb_ref, zs_ref, dinv_ref, bt_ref, b2_ref, wo_ref,
             bo_ref, out_ref):
        c2 = jnp.sum(b2_ref[...] * wo_ref[...])
        tprime = dinv_ref[...] * (ta_ref[...] + tb_ref[...] + zs_ref[...]) + c2
        n = tprime.shape[0]
        seg = lax.broadcasted_iota(jnp.int32, (_G, n), 0)
        oh = (seg == bt_ref[...]).astype(jnp.float32)
        sums = jnp.dot(oh, tprime, preferred_element_type=jnp.float32)
        counts = jnp.dot(oh, jnp.ones((n, 1), jnp.float32),
                         preferred_element_type=jnp.float32)
        out_ref[...] = sums / jnp.maximum(counts, 1.0) + bo_ref[...]

    return pl.pallas_call(
        body,
        out_shape=jax.ShapeDtypeStruct((_G, 1), jnp.float32),
    )(ta, tb, zs, dinv, batch_row, b2r, woutr, bout2d)


def kernel(x, edge_index, edge_weight, batch, W1, b1, W2, b2, Wout, bout):
    n, f = x.shape
    e = edge_weight.shape[0]
    h = W1.shape[1]

    cpt = -(-e // (_C * _NW))
    cpt += (-cpt) % 4
    e_pad = cpt * _C * _NW
    src = edge_index[0]
    dst = edge_index[1]
    w = edge_weight
    if e_pad != e:
        fill = (jnp.arange(e_pad - e, dtype=jnp.int32) % n)
        src = jnp.concatenate([src, fill])
        dst = jnp.concatenate([dst, fill])
        w = jnp.concatenate([w, jnp.zeros((e_pad - e,), w.dtype)])
    n_chunks = e_pad // _C
    src2d = src.reshape(n_chunks, _C)
    dst2d = dst.reshape(n_chunks, _C)
    w2d = w.reshape(n_chunks, _C)

    per_tile = -(-n // _NSUB)
    per_tile += (-per_tile) % _C
    npad = _NSUB * per_tile
    nrows = npad

    degp = _sc_scalar_pass(dst2d, w2d, cpt, npad, n)
    dega = degp[:n].reshape(n, 1)
    degb = degp[npad:npad + n].reshape(n, 1)

    xs, dinv, xsb = _tc_k1(x, W1, dega, degb)

    xsb_p = jax.lax.bitcast_convert_type(
        jnp.take(xsb, _bf16_perm(h), axis=1).reshape(n, h // 2, 2),
        jnp.float32)
    s1p = _sc_row_pass(src2d, dst2d, w2d, xsb_p, cpt * 2, nrows, h)

    zs = _tc_k2(s1p, xs, dinv, b1.reshape(1, h), W2, Wout)

    tp = _sc_scalar_pass(dst2d, w2d, cpt, npad, n,
                         src2d=src2d, zs=zs.reshape(n))
    ta = tp[:n].reshape(n, 1)
    tb = tp[npad:npad + n].reshape(n, 1)

    out = _tc_k3(ta, tb, zs, dinv, batch.reshape(1, n),
                 b2.reshape(1, h), Wout.reshape(1, h),
                 bout.reshape(1, 1))
    return out


# speedup vs baseline: 1.5208x; 1.5208x over previous
import jax
import jax.numpy as jnp
from jax import lax
from jax.experimental import pallas as pl
from jax.experimental.pallas import tpu as pltpu
from jax.experimental.pallas import tpu_sc as plsc

_L = 16
_C = 128
_NSC = 2
_NSUB = 16
_NW = _NSC * _NSUB
_G = 64

_SC_PARAMS = pltpu.CompilerParams(needs_layout_passes=False)
_SC_PARAMS_NT = pltpu.CompilerParams(
    needs_layout_passes=False, use_tc_tiling_on_sc=False)


def _mesh():
    return plsc.VectorSubcoreMesh(core_axis_name="c", subcore_axis_name="s")


def _sc_scalar_pass(dst2d, w2d, cpt, npad, nnodes, src2d=None, zs=None):
    per_tile = npad // _NSUB
    with_gather = zs is not None

    scratch = [
        pltpu.VMEM((cpt, _C), jnp.int32),
        pltpu.VMEM((cpt, _C), jnp.float32),
        pltpu.VMEM((per_tile,), jnp.float32),
        pltpu.VMEM_SHARED((npad,), jnp.float32),
        pltpu.SemaphoreType.DMA,
    ]
    if with_gather:
        scratch.insert(0, pltpu.VMEM((cpt, _C), jnp.int32))
        scratch.insert(1, pltpu.VMEM((nnodes,), jnp.float32))
        scratch.insert(2, pltpu.VMEM((cpt, _C), jnp.float32))

    def body(*refs):
        if with_gather:
            (src_hbm, dst_hbm, w_hbm, zs_hbm, out_hbm,
             srci, zsv, prod, dsti, wv, bnc, acc, sem) = refs
        else:
            (dst_hbm, w_hbm, out_hbm,
             dsti, wv, bnc, acc, sem) = refs
        cid = lax.axis_index("c")
        sid = lax.axis_index("s")
        wid = sid * _NSC + cid
        ofs = wid * cpt

        pltpu.sync_copy(dst_hbm.at[pl.ds(ofs, cpt), :], dsti)
        pltpu.sync_copy(w_hbm.at[pl.ds(ofs, cpt), :], wv)
        if with_gather:
            pltpu.sync_copy(src_hbm.at[pl.ds(ofs, cpt), :], srci)
            pltpu.sync_copy(zs_hbm, zsv)

        zero = jnp.zeros((_L,), jnp.float32)
        for i in range(per_tile // _L):
            bnc[pl.ds(i * _L, _L)] = zero
        pltpu.sync_copy(bnc, acc.at[pl.ds(sid * per_tile, per_tile)])

        if with_gather:
            def pc(c, _):
                for k in range(_C // _L):
                    sl = pl.ds(k * _L, _L)
                    idx = srci[c, sl]
                    prod[c, sl] = plsc.load_gather(zsv, [idx]) * wv[c, sl]
                return 0
            lax.fori_loop(0, cpt, pc, 0)

        plsc.subcore_barrier()

        val = prod if with_gather else wv

        descs = [
            pltpu.async_copy(val.at[c], acc.at[dsti.at[c]], sem, add=True)
            for c in range(cpt)
        ]
        for d in descs:
            d.wait()

        plsc.subcore_barrier()
        pltpu.sync_copy(acc.at[pl.ds(sid * per_tile, per_tile)], bnc)
        pltpu.sync_copy(
            bnc, out_hbm.at[pl.ds(cid * npad + sid * per_tile, per_tile)])

    kern = pl.kernel(
        body,
        mesh=_mesh(),
        out_type=jax.ShapeDtypeStruct((_NSC * npad,), jnp.float32),
        scratch_types=scratch,
        compiler_params=_SC_PARAMS,
    )
    if with_gather:
        return kern(src2d, dst2d, w2d, zs)
    return kern(dst2d, w2d)


def _sc_row_pass(src2d, dst2d, w2d, xs, cpt, nrows, feat):
    rows_per_tile = nrows // _NSUB
    blk = 16
    assert rows_per_tile % _C == 0
    assert cpt % blk == 0

    def body(src_hbm, dst_hbm, w_hbm, xs_hbm, out_hbm,
             srci, dsti, wv, r0, r1, acc, g0, g1, s0, s1):
        rows = [r0, r1]
        gsem = [g0, g1]
        ssem = [s0, s1]
        cid = lax.axis_index("c")
        sid = lax.axis_index("s")
        wid = sid * _NSC + cid
        ofs = wid * cpt
        base = sid * rows_per_tile

        zero = jnp.zeros((_L,), jnp.float32)

        def zr(j, _):
            for k in range(feat // _L):
                r0[j, pl.ds(k * _L, _L)] = zero
            return 0

        lax.fori_loop(0, _C, zr, 0)

        def zb(i, _):
            pltpu.sync_copy(r0, acc.at[pl.ds(base + i * _C, _C), :])
            return 0

        lax.fori_loop(0, rows_per_tile // _C, zb, 0)
        plsc.subcore_barrier()

        def block(b, _):
            bofs = ofs + b * blk
            pltpu.sync_copy(src_hbm.at[pl.ds(bofs, blk), :], srci)
            pltpu.sync_copy(dst_hbm.at[pl.ds(bofs, blk), :], dsti)
            pltpu.sync_copy(w_hbm.at[pl.ds(bofs, blk), :], wv)
            gd = [None, None]
            sd = [None, None]
            gd[0] = pltpu.async_copy(xs_hbm.at[srci.at[0]], rows[0], gsem[0])
            for j in range(blk):
                p = j % 2
                q = 1 - p
                if j + 1 < blk:
                    if sd[q] is not None:
                        sd[q].wait()
                    gd[q] = pltpu.async_copy(
                        xs_hbm.at[srci.at[j + 1]], rows[q], gsem[q])
                gd[p].wait()
                vj = jnp.full((_L,), j, jnp.int32)

                def scale(g, _):
                    e0 = g * _L
                    for jj in range(_L):
                        wj = plsc.load_gather(
                            wv, [vj, jnp.full((_L,), e0 + jj, jnp.int32)])
                        for k in range(feat // _L):
                            sl = pl.ds(k * _L, _L)
                            rows[p][e0 + jj, sl] = rows[p][e0 + jj, sl] * wj
                    return 0

                lax.fori_loop(0, _C // _L, scale, 0)
                sd[p] = pltpu.async_copy(
                    rows[p], acc.at[dsti.at[j]], ssem[p], add=True)
            sd[0].wait()
            sd[1].wait()
            return 0

        lax.fori_loop(0, cpt // blk, block, 0)
        plsc.subcore_barrier()

        for k in range(rows_per_tile // _C):
            sl = pl.ds(base + k * _C, _C)
            pltpu.sync_copy(acc.at[sl, :], r0)
            pltpu.sync_copy(r0, out_hbm.at[cid, sl, :])

    kern = pl.kernel(
        body,
        mesh=_mesh(),
        out_type=jax.ShapeDtypeStruct((_NSC, nrows, feat), jnp.float32),
        scratch_types=(
            [pltpu.VMEM((blk, _C), jnp.int32),
             pltpu.VMEM((blk, _C), jnp.int32),
             pltpu.VMEM((blk, _C), jnp.float32)]
            + [pltpu.VMEM((_C, feat), jnp.float32) for _ in range(2)]
            + [pltpu.VMEM_SHARED((nrows, feat), jnp.float32)]
            + [pltpu.SemaphoreType.DMA for _ in range(4)]
        ),
        compiler_params=_SC_PARAMS,
    )
    return kern(src2d, dst2d, w2d, xs)


def _tc_k1(x, W1, dega, degb):
    def body(x_ref, w_ref, da_ref, db_ref, xs_ref, dinv_ref):
        deg = da_ref[...] + db_ref[...] + 1.0
        dinv = jnp.where(deg > 0,
                         lax.rsqrt(jnp.maximum(deg, 1e-12)),
                         0.0)
        xw = jnp.dot(x_ref[...], w_ref[...],
                     preferred_element_type=jnp.float32)
        xs_ref[...] = xw * dinv
        dinv_ref[...] = dinv

    n, f = x.shape
    h = W1.shape[1]
    return pl.pallas_call(
        body,
        out_shape=[
            jax.ShapeDtypeStruct((n, h), jnp.float32),
            jax.ShapeDtypeStruct((n, 1), jnp.float32),
        ],
    )(x, W1, dega, degb)


def _tc_k2(s1p, xs, dinv, b1r, W2, Wout):
    n = xs.shape[0]

    def body(s_ref, xs_ref, dinv_ref, b1_ref, w2_ref, wo_ref, zs_ref):
        u = jnp.dot(w2_ref[...], wo_ref[...],
                    preferred_element_type=jnp.float32)
        dinv = dinv_ref[...]
        s1 = s_ref[0, :n, :] + s_ref[1, :n, :]
        x1 = dinv * (s1 + xs_ref[...]) + b1_ref[...]
        x1a = 1.0 / (1.0 + jnp.exp(-x1))
        z = jnp.dot(x1a, u, preferred_element_type=jnp.float32)
        zs_ref[...] = z * dinv

    return pl.pallas_call(
        body,
        out_shape=jax.ShapeDtypeStruct((n, 1), jnp.float32),
    )(s1p, xs, dinv, b1r, W2, Wout)


def _tc_k3(ta, tb, zs, dinv, batch_row, b2r, woutr, bout2d):
    def body(ta_ref, tb_ref, zs_ref, dinv_ref, bt_ref, b2_ref, wo_ref,
             bo_ref, out_ref):
        c2 = jnp.sum(b2_ref[...] * wo_ref[...])
        tprime = dinv_ref[...] * (ta_ref[...] + tb_ref[...] + zs_ref[...]) + c2
        n = tprime.shape[0]
        seg = lax.broadcasted_iota(jnp.int32, (_G, n), 0)
        oh = (seg == bt_ref[...]).astype(jnp.float32)
        sums = jnp.dot(oh, tprime, preferred_element_type=jnp.float32)
        counts = jnp.dot(oh, jnp.ones((n, 1), jnp.float32),
                         preferred_element_type=jnp.float32)
        out_ref[...] = sums / jnp.maximum(counts, 1.0) + bo_ref[...]

    return pl.pallas_call(
        body,
        out_shape=jax.ShapeDtypeStruct((_G, 1), jnp.float32),
    )(ta, tb, zs, dinv, batch_row, b2r, woutr, bout2d)


def kernel(x, edge_index, edge_weight, batch, W1, b1, W2, b2, Wout, bout):
    n, f = x.shape
    e = edge_weight.shape[0]
    h = W1.shape[1]

    cpt = -(-e // (_C * _NW))
    cpt += (-cpt) % 4
    e_pad = cpt * _C * _NW
    src = edge_index[0]
    dst = edge_index[1]
    w = edge_weight
    if e_pad != e:
        fill = (jnp.arange(e_pad - e, dtype=jnp.int32) % n)
        src = jnp.concatenate([src, fill])
        dst = jnp.concatenate([dst, fill])
        w = jnp.concatenate([w, jnp.zeros((e_pad - e,), w.dtype)])
    n_chunks = e_pad // _C
    src2d = src.reshape(n_chunks, _C)
    dst2d = dst.reshape(n_chunks, _C)
    w2d = w.reshape(n_chunks, _C)

    per_tile = -(-n // _NSUB)
    per_tile += (-per_tile) % _C
    npad = _NSUB * per_tile
    nrows = npad

    degp = _sc_scalar_pass(dst2d, w2d, cpt, npad, n)
    dega = degp[:n].reshape(n, 1)
    degb = degp[npad:npad + n].reshape(n, 1)

    xs, dinv = _tc_k1(x, W1, dega, degb)

    s1p = _sc_row_pass(src2d, dst2d, w2d, xs, cpt, nrows, h)

    zs = _tc_k2(s1p, xs, dinv, b1.reshape(1, h), W2, Wout)

    tp = _sc_scalar_pass(dst2d, w2d, cpt, npad, n,
                         src2d=src2d, zs=zs.reshape(n))
    ta = tp[:n].reshape(n, 1)
    tb = tp[npad:npad + n].reshape(n, 1)

    out = _tc_k3(ta, tb, zs, dinv, batch.reshape(1, n),
                 b2.reshape(1, h), Wout.reshape(1, h),
                 bout.reshape(1, 1))
    return out
